# baseline (speedup 1.0000x reference)
import jax
import jax.numpy as jnp
from jax import lax
from jax.experimental import pallas as pl
from jax.experimental.pallas import tpu as pltpu

B, S, D_MODEL, D_FEAT = 4, 4096, 2048, 256
TARGET_SNR = 0.3
LN_EPS = 1e-5
CS = 2048
NC = S // CS


def _body(last_ref, ff_ref, g_ref, beta_ref, w_ref, bias_ref, x_ref, out_ref,
          rows_ref, copy_sem, row_sem):
    copies = []
    for b in range(B):
        for k in range(NC):
            cp = pltpu.make_async_copy(
                x_ref.at[pl.ds(b, 1), pl.ds(k * CS, CS), :],
                out_ref.at[pl.ds(b, 1), pl.ds(k * CS, CS), :],
                copy_sem)
            cp.start()
            copies.append(cp)

    row_copies = []
    for b in range(B):
        cp = pltpu.make_async_copy(
            x_ref.at[pl.ds(b, 1), pl.ds(last_ref[b], 1), :],
            rows_ref.at[pl.ds(b, 1)],
            row_sem)
        cp.start()
        row_copies.append(cp)

    ff = ff_ref[...]
    mean = jnp.mean(ff, axis=-1, keepdims=True)
    cent = ff - mean
    var = jnp.mean(cent * cent, axis=-1, keepdims=True)
    ln = cent * lax.rsqrt(var + LN_EPS) * g_ref[...] + beta_ref[...]
    proj = lax.dot_general(ln, w_ref[...], (((1,), (1,)), ((), ())),
                           preferred_element_type=jnp.float32)
    proj = proj + bias_ref[...]
    nrm = jnp.sqrt(jnp.sum(proj * proj, axis=-1, keepdims=True))
    direction = proj / jnp.maximum(nrm, 1e-12)

    for cp in row_copies:
        cp.wait()
    host = rows_ref[...]
    rms = jnp.sqrt(jnp.mean(host * host, axis=-1, keepdims=True))
    rows_ref[...] = host + direction[:, None, :] * (rms * TARGET_SNR)

    for cp in copies:
        cp.wait()
    wb = []
    for b in range(B):
        cp = pltpu.make_async_copy(
            rows_ref.at[pl.ds(b, 1)],
            out_ref.at[pl.ds(b, 1), pl.ds(last_ref[b], 1), :],
            row_sem)
        cp.start()
        wb.append(cp)
    for cp in wb:
        cp.wait()


def kernel(x, faculty_features, ln_gamma, ln_beta, W, b, token_ids,
           last_indices):
    del token_ids
    last = last_indices.astype(jnp.int32)

    return pl.pallas_call(
        _body,
        in_specs=[
            pl.BlockSpec(memory_space=pltpu.SMEM),
            pl.BlockSpec((B, D_FEAT), lambda: (0, 0)),
            pl.BlockSpec((1, D_FEAT), lambda: (0, 0)),
            pl.BlockSpec((1, D_FEAT), lambda: (0, 0)),
            pl.BlockSpec((D_MODEL, D_FEAT), lambda: (0, 0)),
            pl.BlockSpec((1, D_MODEL), lambda: (0, 0)),
            pl.BlockSpec(memory_space=pltpu.MemorySpace.HBM),
        ],
        out_specs=pl.BlockSpec(memory_space=pltpu.MemorySpace.HBM),
        out_shape=jax.ShapeDtypeStruct((B, S, D_MODEL), jnp.float32),
        scratch_shapes=[
            pltpu.VMEM((B, 1, D_MODEL), jnp.float32),
            pltpu.SemaphoreType.DMA,
            pltpu.SemaphoreType.DMA,
        ],
    )(last, faculty_features, ln_gamma.reshape(1, D_FEAT),
      ln_beta.reshape(1, D_FEAT), W, b.reshape(1, D_MODEL), x)


# speedup vs baseline: 47.1760x; 47.1760x over previous
import jax
import jax.numpy as jnp
from jax import lax
from jax.experimental import pallas as pl
from jax.experimental.pallas import tpu as pltpu

B, S, D_MODEL, D_FEAT = 4, 4096, 2048, 256
TARGET_SNR = 0.3
LN_EPS = 1e-5
BS = 1024


def _body(last_ref, x_ref, ff_ref, g_ref, beta_ref, w_ref, bias_ref, out_ref):
    b = pl.program_id(0)
    j = pl.program_id(1)
    last = last_ref[b]
    jb = last // BS
    off = lax.rem(last, BS)

    out_ref[...] = x_ref[...]

    @pl.when(j == jb)
    def _():
        ff = ff_ref[0]
        mean = jnp.mean(ff, axis=-1, keepdims=True)
        cent = ff - mean
        var = jnp.mean(cent * cent, axis=-1, keepdims=True)
        ln = cent * lax.rsqrt(var + LN_EPS) * g_ref[...] + beta_ref[...]
        proj = lax.dot_general(ln, w_ref[...], (((1,), (1,)), ((), ())),
                               preferred_element_type=jnp.float32)
        proj = proj + bias_ref[...]
        nrm = jnp.sqrt(jnp.sum(proj * proj, axis=-1, keepdims=True))
        direction = proj / jnp.maximum(nrm, 1e-12)
        host = x_ref[0, pl.ds(off, 1), :]
        rms = jnp.sqrt(jnp.mean(host * host, axis=-1, keepdims=True))
        out_ref[0, pl.ds(off, 1), :] = host + direction * (rms * TARGET_SNR)


def kernel(x, faculty_features, ln_gamma, ln_beta, W, b, token_ids,
           last_indices):
    del token_ids
    last = last_indices.astype(jnp.int32)

    grid_spec = pltpu.PrefetchScalarGridSpec(
        num_scalar_prefetch=1,
        grid=(B, S // BS),
        in_specs=[
            pl.BlockSpec((1, BS, D_MODEL), lambda i, j, last_ref: (i, j, 0)),
            pl.BlockSpec((1, 1, D_FEAT), lambda i, j, last_ref: (i, 0, 0)),
            pl.BlockSpec((1, D_FEAT), lambda i, j, last_ref: (0, 0)),
            pl.BlockSpec((1, D_FEAT), lambda i, j, last_ref: (0, 0)),
            pl.BlockSpec((D_MODEL, D_FEAT), lambda i, j, last_ref: (0, 0)),
            pl.BlockSpec((1, D_MODEL), lambda i, j, last_ref: (0, 0)),
        ],
        out_specs=pl.BlockSpec((1, BS, D_MODEL),
                               lambda i, j, last_ref: (i, j, 0)),
    )

    return pl.pallas_call(
        _body,
        grid_spec=grid_spec,
        out_shape=jax.ShapeDtypeStruct((B, S, D_MODEL), jnp.float32),
    )(last, x, faculty_features.reshape(B, 1, D_FEAT),
      ln_gamma.reshape(1, D_FEAT), ln_beta.reshape(1, D_FEAT), W,
      b.reshape(1, D_MODEL))
